# baseline (speedup 1.0000x reference)
import functools

import numpy as _np

import jax
import jax.numpy as jnp
from jax import lax
from jax.experimental import pallas as pl
from jax.experimental.pallas import tpu as pltpu
from jax.experimental.pallas import tpu_sc as plsc

H = 128
F_IN = 128
N_T = 10000
V = 10000
E = 150000

V_PAD = 10240
ROWS_PER_TILE = V_PAD // 16
NBLK = 80
BLK = 128
IDXC = 16
E_PAD = 16 * NBLK * BLK
PAD_DST = V
RD_ROWS = 128
N_ROW_BLOCKS = V_PAD // 1024


def _stage_a_body(x_ref, w_ref, b_ref, o_ref, o2_ref):
    t = (jnp.dot(x_ref[...], w_ref[...], preferred_element_type=jnp.float32,
                precision=lax.Precision.HIGHEST)
         + b_ref[...])
    o_ref[...] = t
    o2_ref[...] = t


def _stage_b_body(s1_ref, i1_ref, s2_ref, i2_ref, x0_ref,
                  w1_ref, b1_ref, w2_ref, b2_ref,
                  w3_ref, b3_ref, o1_ref, o2_ref, o3_ref):
    o1_ref[...] = jnp.maximum(
        jnp.dot(s1_ref[...], w1_ref[...], preferred_element_type=jnp.float32,
                precision=lax.Precision.HIGHEST)
        * i1_ref[...] + b1_ref[...], 0.0)
    o2_ref[...] = jnp.maximum(
        jnp.dot(s2_ref[...], w2_ref[...], preferred_element_type=jnp.float32,
                precision=lax.Precision.HIGHEST)
        * i2_ref[...] + b2_ref[...], 0.0)
    o3_ref[...] = jnp.maximum(
        jnp.dot(x0_ref[...], w3_ref[...], preferred_element_type=jnp.float32,
                precision=lax.Precision.HIGHEST)
        + b3_ref[...], 0.0)


def _stage_c_body(sub_ref, iub_ref, sbe_ref, ibe_ref, x1_ref,
                  wub_ref, wbe_ref, b_ref, wr_ref,
                  wc_ref, bc_ref, o_ref):
    t = (jnp.dot(sub_ref[...], wub_ref[...], preferred_element_type=jnp.float32,
                precision=lax.Precision.HIGHEST)
         * iub_ref[...]
         + jnp.dot(sbe_ref[...], wbe_ref[...], preferred_element_type=jnp.float32,
                precision=lax.Precision.HIGHEST)
         * ibe_ref[...]
         + jnp.dot(x1_ref[...], wr_ref[...], preferred_element_type=jnp.float32,
                precision=lax.Precision.HIGHEST)
         + b_ref[...])
    t = jnp.maximum(t, 0.0)
    o_ref[...] = (
        jnp.dot(t, wc_ref[...], preferred_element_type=jnp.float32,
                precision=lax.Precision.HIGHEST) + bc_ref[...]
    )


def _row_spec():
    return pl.BlockSpec((1024, H), lambda i: (i, 0))


def _w_spec():
    return pl.BlockSpec((H, H), lambda i: (0, 0))


def _b_spec():
    return pl.BlockSpec((1, H), lambda i: (0, 0))


def _stage_a(x, w, b):
    return pl.pallas_call(
        _stage_a_body,
        grid=(N_ROW_BLOCKS,),
        in_specs=[_row_spec(), _w_spec(), _b_spec()],
        out_specs=[_row_spec(), _row_spec()],
        out_shape=[jax.ShapeDtypeStruct((V_PAD, H), jnp.float32)] * 2,
    )(x, w, b)


def _inv_spec():
    return pl.BlockSpec((1024, 1), lambda i: (i, 0))


def _stage_b(s1, i1, s2, i2, x0, w1, b1, w2, b2, w3, b3):
    return pl.pallas_call(
        _stage_b_body,
        grid=(N_ROW_BLOCKS,),
        in_specs=[_row_spec(), _inv_spec(), _row_spec(), _inv_spec(),
                  _row_spec(),
                  _w_spec(), _b_spec(), _w_spec(), _b_spec(),
                  _w_spec(), _b_spec()],
        out_specs=[_row_spec(), _row_spec(), _row_spec()],
        out_shape=[jax.ShapeDtypeStruct((V_PAD, H), jnp.float32)] * 3,
    )(s1, i1, s2, i2, x0, w1, b1, w2, b2, w3, b3)


def _stage_c(sub, iub, sbe, ibe, x1, wub, wbe, b, wr, wc, bc):
    return pl.pallas_call(
        _stage_c_body,
        grid=(N_ROW_BLOCKS,),
        in_specs=[_row_spec(), _inv_spec(), _row_spec(), _inv_spec(),
                  _row_spec(),
                  _w_spec(), _w_spec(), _b_spec(), _w_spec(),
                  _w_spec(), _b_spec()],
        out_specs=_row_spec(),
        out_shape=jax.ShapeDtypeStruct((V_PAD, H), jnp.float32),
    )(sub, iub, sbe, ibe, x1, wub, wbe, b, wr, wc, bc)


def _hist_body(d_uc, d_ub, d_he, d_be,
               inv_uc, inv_ub, inv_he, inv_be, dmax,
               cntp, dst_v, cnt_loc, inv_loc, dmax_v):
    c = lax.axis_index("c")
    s = lax.axis_index("s")
    zeros16 = jnp.zeros((16,), jnp.float32)
    ones16 = jnp.ones((16,), jnp.float32)

    def hist_one(dst, inv_out, track_max, dmax_row):
        def zc_body(r, carry):
            cnt_loc[pl.ds(r * 16, 16)] = zeros16
            return carry
        lax.fori_loop(0, V_PAD // 16, zc_body, 0)
        if track_max:
            dmax_v[...] = jnp.full((16,), -1, jnp.int32)

        for r in range(NBLK // IDXC):
            pltpu.sync_copy(dst.at[s, pl.ds(r * IDXC, IDXC)], dst_v)

            def body(j, carry):
                if track_max:
                    dm = dmax_v[...]
                for k in range(8):
                    iv = dst_v[j, pl.ds(k * 16, 16)]
                    plsc.addupdate_scatter(cnt_loc, [iv], ones16)
                    if track_max:
                        dm = jnp.maximum(dm, jnp.where(iv >= PAD_DST, -1, iv))
                if track_max:
                    dmax_v[...] = dm
                return carry
            lax.fori_loop(0, IDXC, body, 0)

        pltpu.sync_copy(cnt_loc, cntp.at[s])
        if track_max:
            pltpu.sync_copy(dmax_v, dmax.at[dmax_row])
        plsc.subcore_barrier()

        base = s * ROWS_PER_TILE
        for i in range(16):
            pltpu.sync_copy(cntp.at[i, pl.ds(base, ROWS_PER_TILE)],
                            cnt_loc.at[pl.ds(i * ROWS_PER_TILE,
                                             ROWS_PER_TILE)])

        def inv_body(k, carry):
            tot = cnt_loc[pl.ds(k * 16, 16)]
            for i in range(1, 16):
                tot = tot + cnt_loc[pl.ds(i * ROWS_PER_TILE + k * 16, 16)]
            inv_loc[pl.ds(k * 16, 16)] = 1.0 / jnp.maximum(tot, 1.0)
            return carry
        lax.fori_loop(0, ROWS_PER_TILE // 16, inv_body, 0)
        pltpu.sync_copy(inv_loc, inv_out.at[pl.ds(base, ROWS_PER_TILE)])
        plsc.subcore_barrier()

    @pl.when(c == 0)
    def _():
        hist_one(d_uc, inv_uc, True, s)
        hist_one(d_ub, inv_ub, False, s)

    @pl.when(c == 1)
    def _():
        hist_one(d_he, inv_he, True, 16 + s)
        hist_one(d_be, inv_be, False, 16 + s)


_HIST_SCRATCH = [
    pltpu.VMEM_SHARED((16, V_PAD), jnp.float32),
    pltpu.VMEM((IDXC, BLK), jnp.int32),
    pltpu.VMEM((V_PAD,), jnp.float32),
    pltpu.VMEM((ROWS_PER_TILE,), jnp.float32),
    pltpu.VMEM((16,), jnp.int32),
]

_HIST_OUT = [
    jax.ShapeDtypeStruct((V_PAD,), jnp.float32),
    jax.ShapeDtypeStruct((V_PAD,), jnp.float32),
    jax.ShapeDtypeStruct((V_PAD,), jnp.float32),
    jax.ShapeDtypeStruct((V_PAD,), jnp.float32),
    jax.ShapeDtypeStruct((32, 16), jnp.int32),
]


def _sc_mesh():
    return plsc.VectorSubcoreMesh(core_axis_name="c", subcore_axis_name="s",
                                  num_cores=2, num_subcores=16)


def _hist(d_uc, d_ub, d_he, d_be):
    fn = pl.kernel(_hist_body, out_type=_HIST_OUT, mesh=_sc_mesh(),
                   scratch_types=_HIST_SCRATCH,
                   compiler_params=pltpu.CompilerParams(
                       needs_layout_passes=False))
    return fn(d_uc, d_ub, d_he, d_be)


NQ = 4
QROWS = BLK // NQ


def _seg_mean_body(tbl, src, dst, clip, sums,
                   acc, src_v, dst_v, rowbuf, bbuf, clip_v, gsem):
    c = lax.axis_index("c")
    s = lax.axis_index("s")
    zeros16 = jnp.zeros((16,), jnp.float32)
    NCH = NBLK // IDXC

    pltpu.sync_copy(clip.at[c], clip_v)

    def z_body(r, carry):
        for k in range(8):
            rowbuf[0, r, pl.ds(k * 16, 16)] = zeros16
        return carry
    lax.fori_loop(0, BLK, z_body, 0)
    for q in range(5):
        pltpu.sync_copy(
            rowbuf.at[0],
            acc.at[pl.ds(s * ROWS_PER_TILE + q * RD_ROWS, RD_ROWS)])

    plsc.subcore_barrier()

    cl = clip_v[...]

    def refill_src(r):
        slot = r % 2
        pltpu.sync_copy(src.at[c, s, pl.ds(r * IDXC, IDXC)], src_v.at[slot])

        def clip_body(j, carry):
            for k in range(8):
                sl = pl.ds(k * 16, 16)
                src_v[slot, j, sl] = jnp.minimum(src_v[slot, j, sl], cl)
            return carry
        lax.fori_loop(0, IDXC, clip_body, 0)

    def fire(g):
        slot, j = (g // IDXC) % 2, g % IDXC
        b = g % 2
        for q in range(NQ):
            pltpu.async_copy(
                tbl.at[c].at[src_v.at[slot, j, pl.ds(q * QROWS, QROWS)]],
                bbuf.at[b, pl.ds(q * QROWS, QROWS)], gsem)

    refill_src(0)
    pltpu.sync_copy(dst.at[c, s, pl.ds(0, IDXC)], dst_v)
    fire(0)
    fire(1)
    for g in range(NBLK):
        r, j = divmod(g, IDXC)
        if j == 0 and g > 0:
            pltpu.sync_copy(dst.at[c, s, pl.ds(r * IDXC, IDXC)], dst_v)
        if j == IDXC - 2 and r + 1 < NCH:
            refill_src(r + 1)
        b = g % 2
        pltpu.make_async_copy(tbl.at[c].at[src_v.at[0, 0]], bbuf.at[b],
                              gsem).wait()

        def conv_body(row, carry):
            for k in range(H // 32):
                v = bbuf[b, row, pl.ds(k * 16, 16)]
                lo = plsc.bitcast(lax.shift_left(v, 16), jnp.float32)
                hi = plsc.bitcast(
                    jnp.bitwise_and(v, jnp.int32(-65536)), jnp.float32)
                rowbuf[0, row, pl.ds(k * 32, 16)] = lo
                rowbuf[0, row, pl.ds(k * 32 + 16, 16)] = hi
            return carry
        lax.fori_loop(0, BLK, conv_body, 0)

        pltpu.sync_copy(rowbuf.at[0], acc.at[dst_v.at[j]], add=True)
        if g + 2 < NBLK:
            fire(g + 2)

    plsc.subcore_barrier()

    base = s * ROWS_PER_TILE
    pltpu.sync_copy(acc.at[pl.ds(base, ROWS_PER_TILE)],
                    sums.at[c, pl.ds(base, ROWS_PER_TILE)])


_SC_SCRATCH = [
    pltpu.VMEM_SHARED((V_PAD, H), jnp.float32),
    pltpu.VMEM((2, IDXC, BLK), jnp.int32),
    pltpu.VMEM((IDXC, BLK), jnp.int32),
    pltpu.VMEM((1, BLK, H), jnp.float32),
    pltpu.VMEM((2, BLK, H // 2), jnp.int32),
    pltpu.VMEM((16,), jnp.int32),
    pltpu.SemaphoreType.DMA,
]

_SC_OUT = [
    jax.ShapeDtypeStruct((2, V_PAD, H), jnp.float32),
]


def _seg_mean(tbl0, src0, dst0, clip0, tbl1, src1, dst1, clip1):
    fn = pl.kernel(_seg_mean_body, out_type=_SC_OUT, mesh=_sc_mesh(),
                   scratch_types=_SC_SCRATCH,
                   compiler_params=pltpu.CompilerParams(
                       needs_layout_passes=False,
                       use_tc_tiling_on_sc=False))
    sums, = fn(jnp.stack([tbl0, tbl1]),
               jnp.stack([src0, src1]),
               jnp.stack([dst0, dst1]),
               jnp.stack([clip0, clip1]))
    return sums[0], sums[1]


def _pad_edges(e):
    per_tile_pad = (E_PAD - E) // 16
    src = jnp.concatenate(
        [e[0].reshape(16, E // 16),
         jnp.zeros((16, per_tile_pad), jnp.int32)], axis=1)
    pad_dst = PAD_DST + (jnp.arange(16 * per_tile_pad, dtype=jnp.int32)
                         % (V_PAD - V)).reshape(16, per_tile_pad)
    dst = jnp.concatenate([e[1].reshape(16, E // 16), pad_dst], axis=1)
    return src.reshape(16, NBLK, BLK), dst.reshape(16, NBLK, BLK)


def kernel(x_transaction, e_uc, e_ub, e_he, e_be, Wt, bt,
           c1_uc_Wl, c1_uc_bl, c1_uc_Wr,
           c1_ub_Wl, c1_ub_bl, c1_ub_Wr,
           c1_he_Wl, c1_he_bl, c1_he_Wr,
           c1_be_Wl, c1_be_bl, c1_be_Wr,
           c2_uc_Wl, c2_uc_bl, c2_uc_Wr,
           c2_ub_Wl, c2_ub_bl, c2_ub_Wr,
           c2_he_Wl, c2_he_bl, c2_he_Wr,
           c2_be_Wl, c2_be_bl, c2_be_Wr,
           Wc, bc):
    xp = jnp.pad(x_transaction, ((0, V_PAD - N_T), (0, 0)))
    x_t0, x_t0b = _stage_a(xp, Wt, bt.reshape(1, H))

    src_uc, dst_uc = _pad_edges(e_uc)
    src_he, dst_he = _pad_edges(e_he)
    src_ub, dst_ub = _pad_edges(e_ub)
    src_be, dst_be = _pad_edges(e_be)

    inv_uc, inv_ub, inv_he, inv_be, dmax1 = _hist(dst_uc, dst_ub,
                                                  dst_he, dst_be)

    clip_const = jnp.full((16,), N_T - 1, jnp.int32)
    def _halfwidth(t):
        return lax.bitcast_convert_type(
            lax.bitcast_convert_type(
                t.astype(jnp.bfloat16).reshape(V_PAD, H // 2, 2),
                jnp.uint32), jnp.int32)

    perm = jnp.asarray(_np.concatenate(
        [g * 32 + _np.concatenate([_np.arange(0, 32, 2), _np.arange(1, 32, 2)])
         for g in range(H // 32)]).astype(_np.int32))

    s_uc, s_he = _seg_mean(_halfwidth(x_t0), src_uc, dst_uc, clip_const,
                           _halfwidth(x_t0b), src_he, dst_he, clip_const)

    x_c1, x_e1, x_t1 = _stage_b(
        s_uc, inv_uc.reshape(V_PAD, 1), s_he, inv_he.reshape(V_PAD, 1), x_t0,
        c1_uc_Wl[perm], c1_uc_bl.reshape(1, H),
        c1_he_Wl[perm], c1_he_bl.reshape(1, H),
        c1_ub_Wr + c1_be_Wr, (c1_ub_bl + c1_be_bl).reshape(1, H))

    clip_ub = jnp.full((16,), jnp.max(dmax1[:16]), jnp.int32)
    clip_be = jnp.full((16,), jnp.max(dmax1[16:]), jnp.int32)
    s_ub, s_be = _seg_mean(_halfwidth(x_c1), src_ub, dst_ub, clip_ub,
                           _halfwidth(x_e1), src_be, dst_be, clip_be)

    wc_pad = jnp.zeros((H, H), jnp.float32).at[:, 0].set(Wc[:, 0])
    bc_pad = jnp.zeros((1, H), jnp.float32).at[0, 0].set(bc[0])
    res = _stage_c(s_ub, inv_ub.reshape(V_PAD, 1),
                   s_be, inv_be.reshape(V_PAD, 1), x_t1,
                   c2_ub_Wl[perm], c2_be_Wl[perm],
                   (c2_ub_bl + c2_be_bl).reshape(1, H),
                   c2_ub_Wr + c2_be_Wr, wc_pad, bc_pad)
    return res[:N_T, 0]


# speedup vs baseline: 1.0121x; 1.0121x over previous
import functools

import numpy as _np

import jax
import jax.numpy as jnp
from jax import lax
from jax.experimental import pallas as pl
from jax.experimental.pallas import tpu as pltpu
from jax.experimental.pallas import tpu_sc as plsc

H = 128
F_IN = 128
N_T = 10000
V = 10000
E = 150000

V_PAD = 10240
ROWS_PER_TILE = V_PAD // 16
NBLK = 80
BLK = 128
IDXC = 16
E_PAD = 16 * NBLK * BLK
PAD_DST = V
RD_ROWS = 128
N_ROW_BLOCKS = V_PAD // 1024


def _stage_a_body(x_ref, w_ref, b_ref, o_ref, o2_ref):
    t = (jnp.dot(x_ref[...], w_ref[...], preferred_element_type=jnp.float32,
                precision=lax.Precision.HIGHEST)
         + b_ref[...])
    o_ref[...] = t
    o2_ref[...] = t


def _stage_b_body(s1_ref, i1_ref, s2_ref, i2_ref, x0_ref,
                  w1_ref, b1_ref, w2_ref, b2_ref,
                  w3_ref, b3_ref, o1_ref, o2_ref, o3_ref):
    o1_ref[...] = jnp.maximum(
        jnp.dot(s1_ref[...], w1_ref[...], preferred_element_type=jnp.float32,
                precision=lax.Precision.HIGHEST)
        * i1_ref[...] + b1_ref[...], 0.0)
    o2_ref[...] = jnp.maximum(
        jnp.dot(s2_ref[...], w2_ref[...], preferred_element_type=jnp.float32,
                precision=lax.Precision.HIGHEST)
        * i2_ref[...] + b2_ref[...], 0.0)
    o3_ref[...] = jnp.maximum(
        jnp.dot(x0_ref[...], w3_ref[...], preferred_element_type=jnp.float32,
                precision=lax.Precision.HIGHEST)
        + b3_ref[...], 0.0)


def _stage_c_body(sub_ref, iub_ref, sbe_ref, ibe_ref, x1_ref,
                  wub_ref, wbe_ref, b_ref, wr_ref,
                  wc_ref, bc_ref, o_ref):
    t = (jnp.dot(sub_ref[...], wub_ref[...], preferred_element_type=jnp.float32,
                precision=lax.Precision.HIGHEST)
         * iub_ref[...]
         + jnp.dot(sbe_ref[...], wbe_ref[...], preferred_element_type=jnp.float32,
                precision=lax.Precision.HIGHEST)
         * ibe_ref[...]
         + jnp.dot(x1_ref[...], wr_ref[...], preferred_element_type=jnp.float32,
                precision=lax.Precision.HIGHEST)
         + b_ref[...])
    t = jnp.maximum(t, 0.0)
    o_ref[...] = (
        jnp.dot(t, wc_ref[...], preferred_element_type=jnp.float32,
                precision=lax.Precision.HIGHEST) + bc_ref[...]
    )


def _row_spec():
    return pl.BlockSpec((1024, H), lambda i: (i, 0))


def _w_spec():
    return pl.BlockSpec((H, H), lambda i: (0, 0))


def _b_spec():
    return pl.BlockSpec((1, H), lambda i: (0, 0))


def _stage_a(x, w, b):
    return pl.pallas_call(
        _stage_a_body,
        grid=(N_ROW_BLOCKS,),
        in_specs=[_row_spec(), _w_spec(), _b_spec()],
        out_specs=[_row_spec(), _row_spec()],
        out_shape=[jax.ShapeDtypeStruct((V_PAD, H), jnp.float32)] * 2,
    )(x, w, b)


def _inv_spec():
    return pl.BlockSpec((1024, 1), lambda i: (i, 0))


def _stage_b(s1, i1, s2, i2, x0, w1, b1, w2, b2, w3, b3):
    return pl.pallas_call(
        _stage_b_body,
        grid=(N_ROW_BLOCKS,),
        in_specs=[_row_spec(), _inv_spec(), _row_spec(), _inv_spec(),
                  _row_spec(),
                  _w_spec(), _b_spec(), _w_spec(), _b_spec(),
                  _w_spec(), _b_spec()],
        out_specs=[_row_spec(), _row_spec(), _row_spec()],
        out_shape=[jax.ShapeDtypeStruct((V_PAD, H), jnp.float32)] * 3,
    )(s1, i1, s2, i2, x0, w1, b1, w2, b2, w3, b3)


def _stage_c(sub, iub, sbe, ibe, x1, wub, wbe, b, wr, wc, bc):
    return pl.pallas_call(
        _stage_c_body,
        grid=(N_ROW_BLOCKS,),
        in_specs=[_row_spec(), _inv_spec(), _row_spec(), _inv_spec(),
                  _row_spec(),
                  _w_spec(), _w_spec(), _b_spec(), _w_spec(),
                  _w_spec(), _b_spec()],
        out_specs=_row_spec(),
        out_shape=jax.ShapeDtypeStruct((V_PAD, H), jnp.float32),
    )(sub, iub, sbe, ibe, x1, wub, wbe, b, wr, wc, bc)


def _hist_body(d_uc, d_ub, d_he, d_be,
               inv_uc, inv_ub, inv_he, inv_be, dmax,
               cntp, dst_v, cnt_loc, inv_loc, dmax_v):
    c = lax.axis_index("c")
    s = lax.axis_index("s")
    zeros16 = jnp.zeros((16,), jnp.float32)
    ones16 = jnp.ones((16,), jnp.float32)

    def hist_one(dst, inv_out, track_max, dmax_row):
        def zc_body(r, carry):
            cnt_loc[pl.ds(r * 16, 16)] = zeros16
            return carry
        lax.fori_loop(0, V_PAD // 16, zc_body, 0)
        if track_max:
            dmax_v[...] = jnp.full((16,), -1, jnp.int32)

        for r in range(NBLK // IDXC):
            pltpu.sync_copy(dst.at[s, pl.ds(r * IDXC, IDXC)], dst_v)

            def body(j, carry):
                if track_max:
                    dm = dmax_v[...]
                for k in range(8):
                    iv = dst_v[j, pl.ds(k * 16, 16)]
                    plsc.addupdate_scatter(cnt_loc, [iv], ones16)
                    if track_max:
                        dm = jnp.maximum(dm, jnp.where(iv >= PAD_DST, -1, iv))
                if track_max:
                    dmax_v[...] = dm
                return carry
            lax.fori_loop(0, IDXC, body, 0)

        pltpu.sync_copy(cnt_loc, cntp.at[s])
        if track_max:
            pltpu.sync_copy(dmax_v, dmax.at[dmax_row])
        plsc.subcore_barrier()

        base = s * ROWS_PER_TILE
        for i in range(16):
            pltpu.sync_copy(cntp.at[i, pl.ds(base, ROWS_PER_TILE)],
                            cnt_loc.at[pl.ds(i * ROWS_PER_TILE,
                                             ROWS_PER_TILE)])

        def inv_body(k, carry):
            tot = cnt_loc[pl.ds(k * 16, 16)]
            for i in range(1, 16):
                tot = tot + cnt_loc[pl.ds(i * ROWS_PER_TILE + k * 16, 16)]
            inv_loc[pl.ds(k * 16, 16)] = 1.0 / jnp.maximum(tot, 1.0)
            return carry
        lax.fori_loop(0, ROWS_PER_TILE // 16, inv_body, 0)
        pltpu.sync_copy(inv_loc, inv_out.at[pl.ds(base, ROWS_PER_TILE)])
        plsc.subcore_barrier()

    @pl.when(c == 0)
    def _():
        hist_one(d_uc, inv_uc, True, s)
        hist_one(d_ub, inv_ub, False, s)

    @pl.when(c == 1)
    def _():
        hist_one(d_he, inv_he, True, 16 + s)
        hist_one(d_be, inv_be, False, 16 + s)


_HIST_SCRATCH = [
    pltpu.VMEM_SHARED((16, V_PAD), jnp.float32),
    pltpu.VMEM((IDXC, BLK), jnp.int32),
    pltpu.VMEM((V_PAD,), jnp.float32),
    pltpu.VMEM((ROWS_PER_TILE,), jnp.float32),
    pltpu.VMEM((16,), jnp.int32),
]

_HIST_OUT = [
    jax.ShapeDtypeStruct((V_PAD,), jnp.float32),
    jax.ShapeDtypeStruct((V_PAD,), jnp.float32),
    jax.ShapeDtypeStruct((V_PAD,), jnp.float32),
    jax.ShapeDtypeStruct((V_PAD,), jnp.float32),
    jax.ShapeDtypeStruct((32, 16), jnp.int32),
]


def _sc_mesh():
    return plsc.VectorSubcoreMesh(core_axis_name="c", subcore_axis_name="s",
                                  num_cores=2, num_subcores=16)


def _hist(d_uc, d_ub, d_he, d_be):
    fn = pl.kernel(_hist_body, out_type=_HIST_OUT, mesh=_sc_mesh(),
                   scratch_types=_HIST_SCRATCH,
                   compiler_params=pltpu.CompilerParams(
                       needs_layout_passes=False))
    return fn(d_uc, d_ub, d_he, d_be)


NQ = 4
QROWS = BLK // NQ


def _seg_mean_body(tbl, src, dst, clip, sums,
                   acc, src_v, dst_v, rowbuf, bbuf, clip_v, gsem):
    c = lax.axis_index("c")
    s = lax.axis_index("s")
    zeros16 = jnp.zeros((16,), jnp.float32)
    NCH = NBLK // IDXC

    pltpu.sync_copy(clip.at[c], clip_v)

    def z_body(r, carry):
        for k in range(8):
            rowbuf[0, r, pl.ds(k * 16, 16)] = zeros16
        return carry
    lax.fori_loop(0, BLK, z_body, 0)
    for q in range(5):
        pltpu.sync_copy(
            rowbuf.at[0],
            acc.at[pl.ds(s * ROWS_PER_TILE + q * RD_ROWS, RD_ROWS)])

    plsc.subcore_barrier()

    cl = clip_v[...]

    def refill_src(r, slot):
        pltpu.sync_copy(src.at[c, s, pl.ds(r * IDXC, IDXC)], src_v.at[slot])

        def clip_body(j, carry):
            for k in range(8):
                sl = pl.ds(k * 16, 16)
                src_v[slot, j, sl] = jnp.minimum(src_v[slot, j, sl], cl)
            return carry
        lax.fori_loop(0, IDXC, clip_body, 0)

    def fire(slot, j, b):
        for q in range(NQ):
            pltpu.async_copy(
                tbl.at[c].at[src_v.at[slot, j, pl.ds(q * QROWS, QROWS)]],
                bbuf.at[b, pl.ds(q * QROWS, QROWS)], gsem)

    refill_src(0, 0)
    pltpu.sync_copy(dst.at[c, s, pl.ds(0, IDXC)], dst_v)
    fire(0, 0, 0)
    fire(0, 1, 1)

    def pair_body(gp, carry):
        for h in (0, 1):
            g = 2 * gp + h
            r = g // IDXC
            j = g % IDXC
            slot_is_1 = (r % 2) == 1

            @pl.when(jnp.logical_and(j == 0, g > 0))
            def _():
                pltpu.sync_copy(dst.at[c, s, pl.ds(r * IDXC, IDXC)], dst_v)

            @pl.when(jnp.logical_and(j == IDXC - 2, r + 1 < NCH))
            def _():
                @pl.when(slot_is_1)
                def _():
                    refill_src(r + 1, 0)

                @pl.when(jnp.logical_not(slot_is_1))
                def _():
                    refill_src(r + 1, 1)

            for q in range(NQ):
                pltpu.make_async_copy(
                    tbl.at[c].at[src_v.at[0, 0, pl.ds(0, QROWS)]],
                    bbuf.at[h, pl.ds(q * QROWS, QROWS)], gsem).wait()

                def conv_body(row, carry2):
                    for k in range(H // 32):
                        v = bbuf[h, row, pl.ds(k * 16, 16)]
                        lo = plsc.bitcast(lax.shift_left(v, 16), jnp.float32)
                        hi = plsc.bitcast(
                            jnp.bitwise_and(v, jnp.int32(-65536)),
                            jnp.float32)
                        rowbuf[0, row, pl.ds(k * 32, 16)] = lo
                        rowbuf[0, row, pl.ds(k * 32 + 16, 16)] = hi
                    return carry2
                lax.fori_loop(q * QROWS, (q + 1) * QROWS, conv_body, 0)

            pltpu.sync_copy(rowbuf.at[0], acc.at[dst_v.at[j]], add=True)

            @pl.when(g + 2 < NBLK)
            def _():
                g2 = g + 2
                slot2_is_1 = ((g2 // IDXC) % 2) == 1

                @pl.when(slot2_is_1)
                def _():
                    fire(1, g2 % IDXC, h)

                @pl.when(jnp.logical_not(slot2_is_1))
                def _():
                    fire(0, g2 % IDXC, h)
        return carry
    lax.fori_loop(0, NBLK // 2, pair_body, 0)

    plsc.subcore_barrier()

    base = s * ROWS_PER_TILE
    pltpu.sync_copy(acc.at[pl.ds(base, ROWS_PER_TILE)],
                    sums.at[c, pl.ds(base, ROWS_PER_TILE)])


_SC_SCRATCH = [
    pltpu.VMEM_SHARED((V_PAD, H), jnp.float32),
    pltpu.VMEM((2, IDXC, BLK), jnp.int32),
    pltpu.VMEM((IDXC, BLK), jnp.int32),
    pltpu.VMEM((1, BLK, H), jnp.float32),
    pltpu.VMEM((2, BLK, H // 2), jnp.int32),
    pltpu.VMEM((16,), jnp.int32),
    pltpu.SemaphoreType.DMA,
]

_SC_OUT = [
    jax.ShapeDtypeStruct((2, V_PAD, H), jnp.float32),
]


def _seg_mean(tbl0, src0, dst0, clip0, tbl1, src1, dst1, clip1):
    fn = pl.kernel(_seg_mean_body, out_type=_SC_OUT, mesh=_sc_mesh(),
                   scratch_types=_SC_SCRATCH,
                   compiler_params=pltpu.CompilerParams(
                       needs_layout_passes=False,
                       use_tc_tiling_on_sc=False))
    sums, = fn(jnp.stack([tbl0, tbl1]),
               jnp.stack([src0, src1]),
               jnp.stack([dst0, dst1]),
               jnp.stack([clip0, clip1]))
    return sums[0], sums[1]


def _pad_edges(e):
    per_tile_pad = (E_PAD - E) // 16
    src = jnp.concatenate(
        [e[0].reshape(16, E // 16),
         jnp.zeros((16, per_tile_pad), jnp.int32)], axis=1)
    pad_dst = PAD_DST + (jnp.arange(16 * per_tile_pad, dtype=jnp.int32)
                         % (V_PAD - V)).reshape(16, per_tile_pad)
    dst = jnp.concatenate([e[1].reshape(16, E // 16), pad_dst], axis=1)
    return src.reshape(16, NBLK, BLK), dst.reshape(16, NBLK, BLK)


def kernel(x_transaction, e_uc, e_ub, e_he, e_be, Wt, bt,
           c1_uc_Wl, c1_uc_bl, c1_uc_Wr,
           c1_ub_Wl, c1_ub_bl, c1_ub_Wr,
           c1_he_Wl, c1_he_bl, c1_he_Wr,
           c1_be_Wl, c1_be_bl, c1_be_Wr,
           c2_uc_Wl, c2_uc_bl, c2_uc_Wr,
           c2_ub_Wl, c2_ub_bl, c2_ub_Wr,
           c2_he_Wl, c2_he_bl, c2_he_Wr,
           c2_be_Wl, c2_be_bl, c2_be_Wr,
           Wc, bc):
    xp = jnp.pad(x_transaction, ((0, V_PAD - N_T), (0, 0)))
    x_t0, x_t0b = _stage_a(xp, Wt, bt.reshape(1, H))

    src_uc, dst_uc = _pad_edges(e_uc)
    src_he, dst_he = _pad_edges(e_he)
    src_ub, dst_ub = _pad_edges(e_ub)
    src_be, dst_be = _pad_edges(e_be)

    inv_uc, inv_ub, inv_he, inv_be, dmax1 = _hist(dst_uc, dst_ub,
                                                  dst_he, dst_be)

    clip_const = jnp.full((16,), N_T - 1, jnp.int32)
    def _halfwidth(t):
        return lax.bitcast_convert_type(
            lax.bitcast_convert_type(
                t.astype(jnp.bfloat16).reshape(V_PAD, H // 2, 2),
                jnp.uint32), jnp.int32)

    perm = jnp.asarray(_np.concatenate(
        [g * 32 + _np.concatenate([_np.arange(0, 32, 2), _np.arange(1, 32, 2)])
         for g in range(H // 32)]).astype(_np.int32))

    s_uc, s_he = _seg_mean(_halfwidth(x_t0), src_uc, dst_uc, clip_const,
                           _halfwidth(x_t0b), src_he, dst_he, clip_const)

    x_c1, x_e1, x_t1 = _stage_b(
        s_uc, inv_uc.reshape(V_PAD, 1), s_he, inv_he.reshape(V_PAD, 1), x_t0,
        c1_uc_Wl[perm], c1_uc_bl.reshape(1, H),
        c1_he_Wl[perm], c1_he_bl.reshape(1, H),
        c1_ub_Wr + c1_be_Wr, (c1_ub_bl + c1_be_bl).reshape(1, H))

    clip_ub = jnp.full((16,), jnp.max(dmax1[:16]), jnp.int32)
    clip_be = jnp.full((16,), jnp.max(dmax1[16:]), jnp.int32)
    s_ub, s_be = _seg_mean(_halfwidth(x_c1), src_ub, dst_ub, clip_ub,
                           _halfwidth(x_e1), src_be, dst_be, clip_be)

    wc_pad = jnp.zeros((H, H), jnp.float32).at[:, 0].set(Wc[:, 0])
    bc_pad = jnp.zeros((1, H), jnp.float32).at[0, 0].set(bc[0])
    res = _stage_c(s_ub, inv_ub.reshape(V_PAD, 1),
                   s_be, inv_be.reshape(V_PAD, 1), x_t1,
                   c2_ub_Wl[perm], c2_be_Wl[perm],
                   (c2_ub_bl + c2_be_bl).reshape(1, H),
                   c2_ub_Wr + c2_be_Wr, wc_pad, bc_pad)
    return res[:N_T, 0]


# speedup vs baseline: 1.1331x; 1.1196x over previous
import functools

import numpy as _np

import jax
import jax.numpy as jnp
from jax import lax
from jax.experimental import pallas as pl
from jax.experimental.pallas import tpu as pltpu
from jax.experimental.pallas import tpu_sc as plsc

H = 128
F_IN = 128
N_T = 10000
V = 10000
E = 150000

V_PAD = 10240
ROWS_PER_TILE = V_PAD // 16
NBLK = 80
BLK = 128
IDXC = 16
E_PAD = 16 * NBLK * BLK
PAD_DST = V
RD_ROWS = 128
N_ROW_BLOCKS = V_PAD // 1024


def _pack_cols(t):
    bits = lax.bitcast_convert_type(t, jnp.int32) + jnp.int32(0x8000)
    left = lax.slice_in_dim(bits, 0, H // 2, axis=1)
    right = lax.slice_in_dim(bits, H // 2, H, axis=1)
    return jnp.bitwise_or(
        lax.shift_right_logical(left, 16),
        jnp.bitwise_and(right, jnp.int32(-65536)))


def _stage_a_body(x_ref, w_ref, b_ref, o_ref, p0_ref, p1_ref):
    t = (jnp.dot(x_ref[...], w_ref[...], preferred_element_type=jnp.float32,
                precision=lax.Precision.HIGHEST)
         + b_ref[...])
    o_ref[...] = t
    p = _pack_cols(t)
    p0_ref[...] = p
    p1_ref[...] = p


def _stage_b_body(s1_ref, i1_ref, s2_ref, i2_ref, x0_ref,
                  w1_ref, b1_ref, w2_ref, b2_ref,
                  w3_ref, b3_ref, pc_ref, pe_ref, o3_ref):
    pc_ref[...] = _pack_cols(jnp.maximum(
        jnp.dot(s1_ref[...], w1_ref[...], preferred_element_type=jnp.float32,
                precision=lax.Precision.HIGHEST)
        * i1_ref[...] + b1_ref[...], 0.0))
    pe_ref[...] = _pack_cols(jnp.maximum(
        jnp.dot(s2_ref[...], w2_ref[...], preferred_element_type=jnp.float32,
                precision=lax.Precision.HIGHEST)
        * i2_ref[...] + b2_ref[...], 0.0))
    o3_ref[...] = jnp.maximum(
        jnp.dot(x0_ref[...], w3_ref[...], preferred_element_type=jnp.float32,
                precision=lax.Precision.HIGHEST)
        + b3_ref[...], 0.0)


def _stage_c_body(sub_ref, iub_ref, sbe_ref, ibe_ref, x1_ref,
                  wub_ref, wbe_ref, b_ref, wr_ref,
                  wc_ref, bc_ref, o_ref):
    t = (jnp.dot(sub_ref[...], wub_ref[...], preferred_element_type=jnp.float32,
                precision=lax.Precision.HIGHEST)
         * iub_ref[...]
         + jnp.dot(sbe_ref[...], wbe_ref[...], preferred_element_type=jnp.float32,
                precision=lax.Precision.HIGHEST)
         * ibe_ref[...]
         + jnp.dot(x1_ref[...], wr_ref[...], preferred_element_type=jnp.float32,
                precision=lax.Precision.HIGHEST)
         + b_ref[...])
    t = jnp.maximum(t, 0.0)
    o_ref[...] = (
        jnp.dot(t, wc_ref[...], preferred_element_type=jnp.float32,
                precision=lax.Precision.HIGHEST) + bc_ref[...]
    )


def _row_spec():
    return pl.BlockSpec((1024, H), lambda i: (i, 0))


def _w_spec():
    return pl.BlockSpec((H, H), lambda i: (0, 0))


def _b_spec():
    return pl.BlockSpec((1, H), lambda i: (0, 0))


def _pack_spec():
    return pl.BlockSpec((1024, H // 2), lambda i: (i, 0))


def _stage_a(x, w, b):
    return pl.pallas_call(
        _stage_a_body,
        grid=(N_ROW_BLOCKS,),
        in_specs=[_row_spec(), _w_spec(), _b_spec()],
        out_specs=[_row_spec(), _pack_spec(), _pack_spec()],
        out_shape=[jax.ShapeDtypeStruct((V_PAD, H), jnp.float32),
                   jax.ShapeDtypeStruct((V_PAD, H // 2), jnp.int32),
                   jax.ShapeDtypeStruct((V_PAD, H // 2), jnp.int32)],
    )(x, w, b)


def _inv_spec():
    return pl.BlockSpec((1024, 1), lambda i: (i, 0))


def _stage_b(s1, i1, s2, i2, x0, w1, b1, w2, b2, w3, b3):
    return pl.pallas_call(
        _stage_b_body,
        grid=(N_ROW_BLOCKS,),
        in_specs=[_row_spec(), _inv_spec(), _row_spec(), _inv_spec(),
                  _row_spec(),
                  _w_spec(), _b_spec(), _w_spec(), _b_spec(),
                  _w_spec(), _b_spec()],
        out_specs=[_pack_spec(), _pack_spec(), _row_spec()],
        out_shape=[jax.ShapeDtypeStruct((V_PAD, H // 2), jnp.int32),
                   jax.ShapeDtypeStruct((V_PAD, H // 2), jnp.int32),
                   jax.ShapeDtypeStruct((V_PAD, H), jnp.float32)],
    )(s1, i1, s2, i2, x0, w1, b1, w2, b2, w3, b3)


def _stage_c(sub, iub, sbe, ibe, x1, wub, wbe, b, wr, wc, bc):
    return pl.pallas_call(
        _stage_c_body,
        grid=(N_ROW_BLOCKS,),
        in_specs=[_row_spec(), _inv_spec(), _row_spec(), _inv_spec(),
                  _row_spec(),
                  _w_spec(), _w_spec(), _b_spec(), _w_spec(),
                  _w_spec(), _b_spec()],
        out_specs=_row_spec(),
        out_shape=jax.ShapeDtypeStruct((V_PAD, H), jnp.float32),
    )(sub, iub, sbe, ibe, x1, wub, wbe, b, wr, wc, bc)


def _hist_body(d_uc, d_ub, d_he, d_be,
               inv_uc, inv_ub, inv_he, inv_be, dmax,
               cntp, dst_v, cnt_loc, inv_loc, dmax_v):
    c = lax.axis_index("c")
    s = lax.axis_index("s")
    zeros16 = jnp.zeros((16,), jnp.float32)
    ones16 = jnp.ones((16,), jnp.float32)

    def hist_one(dst, inv_out, track_max, dmax_row):
        def zc_body(r, carry):
            cnt_loc[pl.ds(r * 16, 16)] = zeros16
            return carry
        lax.fori_loop(0, V_PAD // 16, zc_body, 0)
        if track_max:
            dmax_v[...] = jnp.full((16,), -1, jnp.int32)

        for r in range(NBLK // IDXC):
            pltpu.sync_copy(dst.at[s, pl.ds(r * IDXC, IDXC)], dst_v)

            def body(j, carry):
                if track_max:
                    dm = dmax_v[...]
                for k in range(8):
                    iv = dst_v[j, pl.ds(k * 16, 16)]
                    plsc.addupdate_scatter(cnt_loc, [iv], ones16)
                    if track_max:
                        dm = jnp.maximum(dm, jnp.where(iv >= PAD_DST, -1, iv))
                if track_max:
                    dmax_v[...] = dm
                return carry
            lax.fori_loop(0, IDXC, body, 0)

        pltpu.sync_copy(cnt_loc, cntp.at[s])
        if track_max:
            pltpu.sync_copy(dmax_v, dmax.at[dmax_row])
        plsc.subcore_barrier()

        base = s * ROWS_PER_TILE
        for i in range(16):
            pltpu.sync_copy(cntp.at[i, pl.ds(base, ROWS_PER_TILE)],
                            cnt_loc.at[pl.ds(i * ROWS_PER_TILE,
                                             ROWS_PER_TILE)])

        def inv_body(k, carry):
            tot = cnt_loc[pl.ds(k * 16, 16)]
            for i in range(1, 16):
                tot = tot + cnt_loc[pl.ds(i * ROWS_PER_TILE + k * 16, 16)]
            inv_loc[pl.ds(k * 16, 16)] = 1.0 / jnp.maximum(tot, 1.0)
            return carry
        lax.fori_loop(0, ROWS_PER_TILE // 16, inv_body, 0)
        pltpu.sync_copy(inv_loc, inv_out.at[pl.ds(base, ROWS_PER_TILE)])
        plsc.subcore_barrier()

    @pl.when(c == 0)
    def _():
        hist_one(d_uc, inv_uc, True, s)
        hist_one(d_ub, inv_ub, False, s)

    @pl.when(c == 1)
    def _():
        hist_one(d_he, inv_he, True, 16 + s)
        hist_one(d_be, inv_be, False, 16 + s)


_HIST_SCRATCH = [
    pltpu.VMEM_SHARED((16, V_PAD), jnp.float32),
    pltpu.VMEM((IDXC, BLK), jnp.int32),
    pltpu.VMEM((V_PAD,), jnp.float32),
    pltpu.VMEM((ROWS_PER_TILE,), jnp.float32),
    pltpu.VMEM((16,), jnp.int32),
]

_HIST_OUT = [
    jax.ShapeDtypeStruct((V_PAD,), jnp.float32),
    jax.ShapeDtypeStruct((V_PAD,), jnp.float32),
    jax.ShapeDtypeStruct((V_PAD,), jnp.float32),
    jax.ShapeDtypeStruct((V_PAD,), jnp.float32),
    jax.ShapeDtypeStruct((32, 16), jnp.int32),
]


def _sc_mesh():
    return plsc.VectorSubcoreMesh(core_axis_name="c", subcore_axis_name="s",
                                  num_cores=2, num_subcores=16)


def _hist(d_uc, d_ub, d_he, d_be):
    fn = pl.kernel(_hist_body, out_type=_HIST_OUT, mesh=_sc_mesh(),
                   scratch_types=_HIST_SCRATCH,
                   compiler_params=pltpu.CompilerParams(
                       needs_layout_passes=False))
    return fn(d_uc, d_ub, d_he, d_be)


NQ = 4
QROWS = BLK // NQ


def _seg_mean_body(tbl, src, dst, clip, sums,
                   acc, src_v, dst_v, rowbuf, bbuf, clip_v, gsem):
    c = lax.axis_index("c")
    s = lax.axis_index("s")
    zeros16 = jnp.zeros((16,), jnp.float32)
    NCH = NBLK // IDXC

    pltpu.sync_copy(clip.at[c], clip_v)

    def z_body(r, carry):
        for k in range(8):
            rowbuf[0, r, pl.ds(k * 16, 16)] = zeros16
        return carry
    lax.fori_loop(0, BLK, z_body, 0)
    for q in range(5):
        pltpu.sync_copy(
            rowbuf.at[0],
            acc.at[pl.ds(s * ROWS_PER_TILE + q * RD_ROWS, RD_ROWS)])

    plsc.subcore_barrier()

    cl = clip_v[...]

    def refill_src(r, slot):
        pltpu.sync_copy(src.at[c, s, pl.ds(r * IDXC, IDXC)], src_v.at[slot])

        def clip_body(j, carry):
            for k in range(8):
                sl = pl.ds(k * 16, 16)
                src_v[slot, j, sl] = jnp.minimum(src_v[slot, j, sl], cl)
            return carry
        lax.fori_loop(0, IDXC, clip_body, 0)

    def fire(slot, j, b):
        for q in range(NQ):
            pltpu.async_copy(
                tbl.at[c].at[src_v.at[slot, j, pl.ds(q * QROWS, QROWS)]],
                bbuf.at[b, pl.ds(q * QROWS, QROWS)], gsem)

    refill_src(0, 0)
    pltpu.sync_copy(dst.at[c, s, pl.ds(0, IDXC)], dst_v)
    fire(0, 0, 0)
    fire(0, 1, 1)

    def pair_body(gp, carry):
        for h in (0, 1):
            g = 2 * gp + h
            r = g // IDXC
            j = g % IDXC
            slot_is_1 = (r % 2) == 1

            @pl.when(jnp.logical_and(j == 0, g > 0))
            def _():
                pltpu.sync_copy(dst.at[c, s, pl.ds(r * IDXC, IDXC)], dst_v)

            @pl.when(jnp.logical_and(j == IDXC - 2, r + 1 < NCH))
            def _():
                @pl.when(slot_is_1)
                def _():
                    refill_src(r + 1, 0)

                @pl.when(jnp.logical_not(slot_is_1))
                def _():
                    refill_src(r + 1, 1)

            for q in range(NQ):
                pltpu.make_async_copy(
                    tbl.at[c].at[src_v.at[0, 0, pl.ds(0, QROWS)]],
                    bbuf.at[h, pl.ds(q * QROWS, QROWS)], gsem).wait()

                def conv_body(row, carry2):
                    for k in range(H // 32):
                        v = bbuf[h, row, pl.ds(k * 16, 16)]
                        lo = plsc.bitcast(lax.shift_left(v, 16), jnp.float32)
                        hi = plsc.bitcast(
                            jnp.bitwise_and(v, jnp.int32(-65536)),
                            jnp.float32)
                        rowbuf[0, row, pl.ds(k * 16, 16)] = lo
                        rowbuf[0, row, pl.ds(H // 2 + k * 16, 16)] = hi
                    return carry2
                lax.fori_loop(q * QROWS, (q + 1) * QROWS, conv_body, 0)

            pltpu.sync_copy(rowbuf.at[0], acc.at[dst_v.at[j]], add=True)

            @pl.when(g + 2 < NBLK)
            def _():
                g2 = g + 2
                slot2_is_1 = ((g2 // IDXC) % 2) == 1

                @pl.when(slot2_is_1)
                def _():
                    fire(1, g2 % IDXC, h)

                @pl.when(jnp.logical_not(slot2_is_1))
                def _():
                    fire(0, g2 % IDXC, h)
        return carry
    lax.fori_loop(0, NBLK // 2, pair_body, 0)

    plsc.subcore_barrier()

    base = s * ROWS_PER_TILE
    pltpu.sync_copy(acc.at[pl.ds(base, ROWS_PER_TILE)],
                    sums.at[c, pl.ds(base, ROWS_PER_TILE)])


_SC_SCRATCH = [
    pltpu.VMEM_SHARED((V_PAD, H), jnp.float32),
    pltpu.VMEM((2, IDXC, BLK), jnp.int32),
    pltpu.VMEM((IDXC, BLK), jnp.int32),
    pltpu.VMEM((1, BLK, H), jnp.float32),
    pltpu.VMEM((2, BLK, H // 2), jnp.int32),
    pltpu.VMEM((16,), jnp.int32),
    pltpu.SemaphoreType.DMA,
]

_SC_OUT = [
    jax.ShapeDtypeStruct((2, V_PAD, H), jnp.float32),
]


def _seg_mean(tbl0, src0, dst0, clip0, tbl1, src1, dst1, clip1):
    fn = pl.kernel(_seg_mean_body, out_type=_SC_OUT, mesh=_sc_mesh(),
                   scratch_types=_SC_SCRATCH,
                   compiler_params=pltpu.CompilerParams(
                       needs_layout_passes=False,
                       use_tc_tiling_on_sc=False))
    sums, = fn(jnp.stack([tbl0, tbl1]),
               jnp.stack([src0, src1]),
               jnp.stack([dst0, dst1]),
               jnp.stack([clip0, clip1]))
    return sums[0], sums[1]


def _pad_edges(e):
    per_tile_pad = (E_PAD - E) // 16
    src = jnp.concatenate(
        [e[0].reshape(16, E // 16),
         jnp.zeros((16, per_tile_pad), jnp.int32)], axis=1)
    pad_dst = PAD_DST + (jnp.arange(16 * per_tile_pad, dtype=jnp.int32)
                         % (V_PAD - V)).reshape(16, per_tile_pad)
    dst = jnp.concatenate([e[1].reshape(16, E // 16), pad_dst], axis=1)
    return src.reshape(16, NBLK, BLK), dst.reshape(16, NBLK, BLK)


def kernel(x_transaction, e_uc, e_ub, e_he, e_be, Wt, bt,
           c1_uc_Wl, c1_uc_bl, c1_uc_Wr,
           c1_ub_Wl, c1_ub_bl, c1_ub_Wr,
           c1_he_Wl, c1_he_bl, c1_he_Wr,
           c1_be_Wl, c1_be_bl, c1_be_Wr,
           c2_uc_Wl, c2_uc_bl, c2_uc_Wr,
           c2_ub_Wl, c2_ub_bl, c2_ub_Wr,
           c2_he_Wl, c2_he_bl, c2_he_Wr,
           c2_be_Wl, c2_be_bl, c2_be_Wr,
           Wc, bc):
    xp = jnp.pad(x_transaction, ((0, V_PAD - N_T), (0, 0)))
    x_t0, p_t0a, p_t0b = _stage_a(xp, Wt, bt.reshape(1, H))

    src_uc, dst_uc = _pad_edges(e_uc)
    src_he, dst_he = _pad_edges(e_he)
    src_ub, dst_ub = _pad_edges(e_ub)
    src_be, dst_be = _pad_edges(e_be)

    inv_uc, inv_ub, inv_he, inv_be, dmax1 = _hist(dst_uc, dst_ub,
                                                  dst_he, dst_be)

    clip_const = jnp.full((16,), N_T - 1, jnp.int32)
    s_uc, s_he = _seg_mean(p_t0a, src_uc, dst_uc, clip_const,
                           p_t0b, src_he, dst_he, clip_const)

    p_c1, p_e1, x_t1 = _stage_b(
        s_uc, inv_uc.reshape(V_PAD, 1), s_he, inv_he.reshape(V_PAD, 1), x_t0,
        c1_uc_Wl, c1_uc_bl.reshape(1, H),
        c1_he_Wl, c1_he_bl.reshape(1, H),
        c1_ub_Wr + c1_be_Wr, (c1_ub_bl + c1_be_bl).reshape(1, H))

    clip_ub = jnp.full((16,), jnp.max(dmax1[:16]), jnp.int32)
    clip_be = jnp.full((16,), jnp.max(dmax1[16:]), jnp.int32)
    s_ub, s_be = _seg_mean(p_c1, src_ub, dst_ub, clip_ub,
                           p_e1, src_be, dst_be, clip_be)

    wc_pad = jnp.zeros((H, H), jnp.float32).at[:, 0].set(Wc[:, 0])
    bc_pad = jnp.zeros((1, H), jnp.float32).at[0, 0].set(bc[0])
    res = _stage_c(s_ub, inv_ub.reshape(V_PAD, 1),
                   s_be, inv_be.reshape(V_PAD, 1), x_t1,
                   c2_ub_Wl, c2_be_Wl,
                   (c2_ub_bl + c2_be_bl).reshape(1, H),
                   c2_ub_Wr + c2_be_Wr, wc_pad, bc_pad)
    return res[:N_T, 0]


# speedup vs baseline: 1.1335x; 1.0004x over previous
import jax
import jax.numpy as jnp
from jax import lax
from jax.experimental import pallas as pl
from jax.experimental.pallas import tpu as pltpu
from jax.experimental.pallas import tpu_sc as plsc

H = 128
F_IN = 128
N_T = 10000
V = 10000
E = 150000

V_PAD = 10240
ROWS_PER_TILE = V_PAD // 16
NBLK = 80
BLK = 128
IDXC = 16
E_PAD = 16 * NBLK * BLK
PAD_DST = V
RD_ROWS = 128
N_ROW_BLOCKS = V_PAD // 1024


def _pack_cols(t):
    bits = lax.bitcast_convert_type(t, jnp.int32) + jnp.int32(0x8000)
    left = lax.slice_in_dim(bits, 0, H // 2, axis=1)
    right = lax.slice_in_dim(bits, H // 2, H, axis=1)
    return jnp.bitwise_or(
        lax.shift_right_logical(left, 16),
        jnp.bitwise_and(right, jnp.int32(-65536)))


def _stage_a_body(x_ref, w_ref, b_ref, o_ref, p0_ref, p1_ref):
    t = (jnp.dot(x_ref[...], w_ref[...], preferred_element_type=jnp.float32,
                precision=lax.Precision.HIGHEST)
         + b_ref[...])
    o_ref[...] = t
    p = _pack_cols(t)
    p0_ref[...] = p
    p1_ref[...] = p


def _stage_b_body(s1_ref, i1_ref, s2_ref, i2_ref, x0_ref,
                  w1_ref, b1_ref, w2_ref, b2_ref,
                  w3_ref, b3_ref, pc_ref, pe_ref, o3_ref):
    pc_ref[...] = _pack_cols(jnp.maximum(
        jnp.dot(s1_ref[...], w1_ref[...], preferred_element_type=jnp.float32,
                precision=lax.Precision.HIGHEST)
        * i1_ref[...] + b1_ref[...], 0.0))
    pe_ref[...] = _pack_cols(jnp.maximum(
        jnp.dot(s2_ref[...], w2_ref[...], preferred_element_type=jnp.float32,
                precision=lax.Precision.HIGHEST)
        * i2_ref[...] + b2_ref[...], 0.0))
    o3_ref[...] = jnp.maximum(
        jnp.dot(x0_ref[...], w3_ref[...], preferred_element_type=jnp.float32,
                precision=lax.Precision.HIGHEST)
        + b3_ref[...], 0.0)


def _stage_c_body(sub_ref, iub_ref, sbe_ref, ibe_ref, x1_ref,
                  wub_ref, wbe_ref, b_ref, wr_ref,
                  wc_ref, bc_ref, o_ref):
    t = (jnp.dot(sub_ref[...], wub_ref[...], preferred_element_type=jnp.float32,
                precision=lax.Precision.HIGHEST)
         * iub_ref[...]
         + jnp.dot(sbe_ref[...], wbe_ref[...], preferred_element_type=jnp.float32,
                precision=lax.Precision.HIGHEST)
         * ibe_ref[...]
         + jnp.dot(x1_ref[...], wr_ref[...], preferred_element_type=jnp.float32,
                precision=lax.Precision.HIGHEST)
         + b_ref[...])
    t = jnp.maximum(t, 0.0)
    o_ref[...] = (
        jnp.dot(t, wc_ref[...], preferred_element_type=jnp.float32,
                precision=lax.Precision.HIGHEST) + bc_ref[...]
    )


def _row_spec():
    return pl.BlockSpec((1024, H), lambda i: (i, 0))


def _w_spec():
    return pl.BlockSpec((H, H), lambda i: (0, 0))


def _b_spec():
    return pl.BlockSpec((1, H), lambda i: (0, 0))


def _pack_spec():
    return pl.BlockSpec((1024, H // 2), lambda i: (i, 0))


def _stage_a(x, w, b):
    return pl.pallas_call(
        _stage_a_body,
        grid=(N_ROW_BLOCKS,),
        in_specs=[_row_spec(), _w_spec(), _b_spec()],
        out_specs=[_row_spec(), _pack_spec(), _pack_spec()],
        out_shape=[jax.ShapeDtypeStruct((V_PAD, H), jnp.float32),
                   jax.ShapeDtypeStruct((V_PAD, H // 2), jnp.int32),
                   jax.ShapeDtypeStruct((V_PAD, H // 2), jnp.int32)],
    )(x, w, b)


def _inv_spec():
    return pl.BlockSpec((1024, 1), lambda i: (i, 0))


def _stage_b(s1, i1, s2, i2, x0, w1, b1, w2, b2, w3, b3):
    return pl.pallas_call(
        _stage_b_body,
        grid=(N_ROW_BLOCKS,),
        in_specs=[_row_spec(), _inv_spec(), _row_spec(), _inv_spec(),
                  _row_spec(),
                  _w_spec(), _b_spec(), _w_spec(), _b_spec(),
                  _w_spec(), _b_spec()],
        out_specs=[_pack_spec(), _pack_spec(), _row_spec()],
        out_shape=[jax.ShapeDtypeStruct((V_PAD, H // 2), jnp.int32),
                   jax.ShapeDtypeStruct((V_PAD, H // 2), jnp.int32),
                   jax.ShapeDtypeStruct((V_PAD, H), jnp.float32)],
    )(s1, i1, s2, i2, x0, w1, b1, w2, b2, w3, b3)


def _stage_c(sub, iub, sbe, ibe, x1, wub, wbe, b, wr, wc, bc):
    return pl.pallas_call(
        _stage_c_body,
        grid=(N_ROW_BLOCKS,),
        in_specs=[_row_spec(), _inv_spec(), _row_spec(), _inv_spec(),
                  _row_spec(),
                  _w_spec(), _w_spec(), _b_spec(), _w_spec(),
                  _w_spec(), _b_spec()],
        out_specs=_row_spec(),
        out_shape=jax.ShapeDtypeStruct((V_PAD, H), jnp.float32),
    )(sub, iub, sbe, ibe, x1, wub, wbe, b, wr, wc, bc)


def _hist_body(d_uc, d_ub, d_he, d_be,
               inv_uc, inv_ub, inv_he, inv_be, dmax,
               cntp, dst_v, cnt_loc, inv_loc, dmax_v):
    c = lax.axis_index("c")
    s = lax.axis_index("s")
    zeros16 = jnp.zeros((16,), jnp.float32)
    ones16 = jnp.ones((16,), jnp.float32)

    def hist_one(dst, inv_out, track_max, dmax_row):
        def zc_body(r, carry):
            cnt_loc[pl.ds(r * 16, 16)] = zeros16
            return carry
        lax.fori_loop(0, V_PAD // 16, zc_body, 0)
        if track_max:
            dmax_v[...] = jnp.full((16,), -1, jnp.int32)

        for r in range(NBLK // IDXC):
            pltpu.sync_copy(dst.at[s, pl.ds(r * IDXC, IDXC)], dst_v)

            def body(j, carry):
                if track_max:
                    dm = dmax_v[...]
                for k in range(8):
                    iv = dst_v[j, pl.ds(k * 16, 16)]
                    plsc.addupdate_scatter(cnt_loc, [iv], ones16)
                    if track_max:
                        dm = jnp.maximum(dm, jnp.where(iv >= PAD_DST, -1, iv))
                if track_max:
                    dmax_v[...] = dm
                return carry
            lax.fori_loop(0, IDXC, body, 0)

        pltpu.sync_copy(cnt_loc, cntp.at[s])
        if track_max:
            pltpu.sync_copy(dmax_v, dmax.at[dmax_row])
        plsc.subcore_barrier()

        base = s * ROWS_PER_TILE
        for i in range(16):
            pltpu.sync_copy(cntp.at[i, pl.ds(base, ROWS_PER_TILE)],
                            cnt_loc.at[pl.ds(i * ROWS_PER_TILE,
                                             ROWS_PER_TILE)])

        def inv_body(k, carry):
            tot = cnt_loc[pl.ds(k * 16, 16)]
            for i in range(1, 16):
                tot = tot + cnt_loc[pl.ds(i * ROWS_PER_TILE + k * 16, 16)]
            inv_loc[pl.ds(k * 16, 16)] = 1.0 / jnp.maximum(tot, 1.0)
            return carry
        lax.fori_loop(0, ROWS_PER_TILE // 16, inv_body, 0)
        pltpu.sync_copy(inv_loc, inv_out.at[pl.ds(base, ROWS_PER_TILE)])
        plsc.subcore_barrier()

    @pl.when(c == 0)
    def _():
        hist_one(d_uc, inv_uc, True, s)
        hist_one(d_ub, inv_ub, False, s)

    @pl.when(c == 1)
    def _():
        hist_one(d_he, inv_he, True, 16 + s)
        hist_one(d_be, inv_be, False, 16 + s)


_HIST_SCRATCH = [
    pltpu.VMEM_SHARED((16, V_PAD), jnp.float32),
    pltpu.VMEM((IDXC, BLK), jnp.int32),
    pltpu.VMEM((V_PAD,), jnp.float32),
    pltpu.VMEM((ROWS_PER_TILE,), jnp.float32),
    pltpu.VMEM((16,), jnp.int32),
]

_HIST_OUT = [
    jax.ShapeDtypeStruct((V_PAD,), jnp.float32),
    jax.ShapeDtypeStruct((V_PAD,), jnp.float32),
    jax.ShapeDtypeStruct((V_PAD,), jnp.float32),
    jax.ShapeDtypeStruct((V_PAD,), jnp.float32),
    jax.ShapeDtypeStruct((32, 16), jnp.int32),
]


def _sc_mesh():
    return plsc.VectorSubcoreMesh(core_axis_name="c", subcore_axis_name="s",
                                  num_cores=2, num_subcores=16)


def _hist(d_uc, d_ub, d_he, d_be):
    fn = pl.kernel(_hist_body, out_type=_HIST_OUT, mesh=_sc_mesh(),
                   scratch_types=_HIST_SCRATCH,
                   compiler_params=pltpu.CompilerParams(
                       needs_layout_passes=False))
    return fn(d_uc, d_ub, d_he, d_be)


NQ = 4
QROWS = BLK // NQ


def _seg_mean_body(tbl, src, dst, clip, sums,
                   acc, src_v, dst_v, rowbuf, bbuf, clip_v, gsem):
    c = lax.axis_index("c")
    s = lax.axis_index("s")
    zeros16 = jnp.zeros((16,), jnp.float32)
    NCH = NBLK // IDXC

    pltpu.sync_copy(clip.at[c], clip_v)

    def z_body(r, carry):
        for k in range(8):
            rowbuf[0, r, pl.ds(k * 16, 16)] = zeros16
        return carry
    lax.fori_loop(0, BLK, z_body, 0)
    for q in range(5):
        pltpu.sync_copy(
            rowbuf.at[0],
            acc.at[pl.ds(s * ROWS_PER_TILE + q * RD_ROWS, RD_ROWS)])

    plsc.subcore_barrier()

    cl = clip_v[...]

    def refill_src(r, slot):
        pltpu.sync_copy(src.at[c, s, pl.ds(r * IDXC, IDXC)], src_v.at[slot])

        def clip_body(j, carry):
            for k in range(8):
                sl = pl.ds(k * 16, 16)
                src_v[slot, j, sl] = jnp.minimum(src_v[slot, j, sl], cl)
            return carry
        lax.fori_loop(0, IDXC, clip_body, 0)

    def fire(slot, j, b):
        for q in range(NQ):
            pltpu.async_copy(
                tbl.at[c].at[src_v.at[slot, j, pl.ds(q * QROWS, QROWS)]],
                bbuf.at[b, pl.ds(q * QROWS, QROWS)], gsem)

    refill_src(0, 0)
    pltpu.sync_copy(dst.at[c, s, pl.ds(0, IDXC)], dst_v)
    fire(0, 0, 0)
    fire(0, 1, 1)

    def pair_body(gp, carry):
        for h in (0, 1):
            g = 2 * gp + h
            r = g // IDXC
            j = g % IDXC
            slot_is_1 = (r % 2) == 1

            @pl.when(jnp.logical_and(j == 0, g > 0))
            def _():
                pltpu.sync_copy(dst.at[c, s, pl.ds(r * IDXC, IDXC)], dst_v)

            @pl.when(jnp.logical_and(j == IDXC - 2, r + 1 < NCH))
            def _():
                @pl.when(slot_is_1)
                def _():
                    refill_src(r + 1, 0)

                @pl.when(jnp.logical_not(slot_is_1))
                def _():
                    refill_src(r + 1, 1)

            for q in range(NQ):
                pltpu.make_async_copy(
                    tbl.at[c].at[src_v.at[0, 0, pl.ds(0, QROWS)]],
                    bbuf.at[h, pl.ds(q * QROWS, QROWS)], gsem).wait()

                def conv_body(row, carry2):
                    for k in range(H // 32):
                        v = bbuf[h, row, pl.ds(k * 16, 16)]
                        lo = plsc.bitcast(lax.shift_left(v, 16), jnp.float32)
                        hi = plsc.bitcast(
                            jnp.bitwise_and(v, jnp.int32(-65536)),
                            jnp.float32)
                        rowbuf[0, row, pl.ds(k * 16, 16)] = lo
                        rowbuf[0, row, pl.ds(H // 2 + k * 16, 16)] = hi
                    return carry2
                lax.fori_loop(q * QROWS, (q + 1) * QROWS, conv_body, 0)

            pltpu.sync_copy(rowbuf.at[0], acc.at[dst_v.at[j]], add=True)

            @pl.when(g + 2 < NBLK)
            def _():
                g2 = g + 2
                slot2_is_1 = ((g2 // IDXC) % 2) == 1

                @pl.when(slot2_is_1)
                def _():
                    fire(1, g2 % IDXC, h)

                @pl.when(jnp.logical_not(slot2_is_1))
                def _():
                    fire(0, g2 % IDXC, h)
        return carry
    lax.fori_loop(0, NBLK // 2, pair_body, 0)

    plsc.subcore_barrier()

    base = s * ROWS_PER_TILE
    pltpu.sync_copy(acc.at[pl.ds(base, ROWS_PER_TILE)],
                    sums.at[c, pl.ds(base, ROWS_PER_TILE)])


_SC_SCRATCH = [
    pltpu.VMEM_SHARED((V_PAD, H), jnp.float32),
    pltpu.VMEM((2, IDXC, BLK), jnp.int32),
    pltpu.VMEM((IDXC, BLK), jnp.int32),
    pltpu.VMEM((1, BLK, H), jnp.float32),
    pltpu.VMEM((2, BLK, H // 2), jnp.int32),
    pltpu.VMEM((16,), jnp.int32),
    pltpu.SemaphoreType.DMA,
]

_SC_OUT = [
    jax.ShapeDtypeStruct((2, V_PAD, H), jnp.float32),
]


def _seg_mean(tbl0, src0, dst0, clip0, tbl1, src1, dst1, clip1):
    fn = pl.kernel(_seg_mean_body, out_type=_SC_OUT, mesh=_sc_mesh(),
                   scratch_types=_SC_SCRATCH,
                   compiler_params=pltpu.CompilerParams(
                       needs_layout_passes=False,
                       use_tc_tiling_on_sc=False))
    sums, = fn(jnp.stack([tbl0, tbl1]),
               jnp.stack([src0, src1]),
               jnp.stack([dst0, dst1]),
               jnp.stack([clip0, clip1]))
    return sums[0], sums[1]


def _pad_edges(e):
    per_tile_pad = (E_PAD - E) // 16
    src = jnp.concatenate(
        [e[0].reshape(16, E // 16),
         jnp.zeros((16, per_tile_pad), jnp.int32)], axis=1)
    pad_dst = PAD_DST + (jnp.arange(16 * per_tile_pad, dtype=jnp.int32)
                         % (V_PAD - V)).reshape(16, per_tile_pad)
    dst = jnp.concatenate([e[1].reshape(16, E // 16), pad_dst], axis=1)
    return src.reshape(16, NBLK, BLK), dst.reshape(16, NBLK, BLK)


def kernel(x_transaction, e_uc, e_ub, e_he, e_be, Wt, bt,
           c1_uc_Wl, c1_uc_bl, c1_uc_Wr,
           c1_ub_Wl, c1_ub_bl, c1_ub_Wr,
           c1_he_Wl, c1_he_bl, c1_he_Wr,
           c1_be_Wl, c1_be_bl, c1_be_Wr,
           c2_uc_Wl, c2_uc_bl, c2_uc_Wr,
           c2_ub_Wl, c2_ub_bl, c2_ub_Wr,
           c2_he_Wl, c2_he_bl, c2_he_Wr,
           c2_be_Wl, c2_be_bl, c2_be_Wr,
           Wc, bc):
    xp = jnp.pad(x_transaction, ((0, V_PAD - N_T), (0, 0)))
    x_t0, p_t0a, p_t0b = _stage_a(xp, Wt, bt.reshape(1, H))

    src_uc, dst_uc = _pad_edges(e_uc)
    src_he, dst_he = _pad_edges(e_he)
    src_ub, dst_ub = _pad_edges(e_ub)
    src_be, dst_be = _pad_edges(e_be)

    inv_uc, inv_ub, inv_he, inv_be, dmax1 = _hist(dst_uc, dst_ub,
                                                  dst_he, dst_be)

    clip_const = jnp.full((16,), N_T - 1, jnp.int32)
    s_uc, s_he = _seg_mean(p_t0a, src_uc, dst_uc, clip_const,
                           p_t0b, src_he, dst_he, clip_const)

    p_c1, p_e1, x_t1 = _stage_b(
        s_uc, inv_uc.reshape(V_PAD, 1), s_he, inv_he.reshape(V_PAD, 1), x_t0,
        c1_uc_Wl, c1_uc_bl.reshape(1, H),
        c1_he_Wl, c1_he_bl.reshape(1, H),
        c1_ub_Wr + c1_be_Wr, (c1_ub_bl + c1_be_bl).reshape(1, H))

    clip_ub = jnp.full((16,), jnp.max(dmax1[:16]), jnp.int32)
    clip_be = jnp.full((16,), jnp.max(dmax1[16:]), jnp.int32)
    s_ub, s_be = _seg_mean(p_c1, src_ub, dst_ub, clip_ub,
                           p_e1, src_be, dst_be, clip_be)

    wc_pad = jnp.zeros((H, H), jnp.float32).at[:, 0].set(Wc[:, 0])
    bc_pad = jnp.zeros((1, H), jnp.float32).at[0, 0].set(bc[0])
    res = _stage_c(s_ub, inv_ub.reshape(V_PAD, 1),
                   s_be, inv_be.reshape(V_PAD, 1), x_t1,
                   c2_ub_Wl, c2_be_Wl,
                   (c2_ub_bl + c2_be_bl).reshape(1, H),
                   c2_ub_Wr + c2_be_Wr, wc_pad, bc_pad)
    return res[:N_T, 0]
